# baseline (speedup 1.0000x reference)
import functools

import jax
import jax.numpy as jnp
from jax import lax
from jax.experimental import pallas as pl
from jax.experimental.pallas import tpu as pltpu
from jax.experimental.pallas import tpu_sc as plsc

_B = 128
_N = 8192
_L = 16
_NC = 2
_NS = 16
_NW = _NC * _NS
_RPW = _B // _NW
_U = 4
_NBLK = _N // _L
_NIT = _NBLK // _U


def _sc_body(coords_hbm, out_hbm, row_v, oh_v):
    wid = lax.axis_index("s") * _NC + lax.axis_index("c")
    lane = lax.broadcasted_iota(jnp.int32, (_L,), 0)
    zeros16 = jnp.zeros((_L,), jnp.float32)

    def zinit(i, c):
        oh_v[pl.ds(i * _L, _L)] = zeros16
        return c

    lax.fori_loop(0, _NBLK, zinit, 0)

    for r in range(_RPW):
        row = wid * _RPW + r
        pltpu.sync_copy(coords_hbm.at[row], row_v)

        neg = jnp.full((_L,), -jnp.inf, jnp.float32)
        iz = jnp.zeros((_L,), jnp.int32)

        def step(i, carry):
            bvs, bis = carry
            nbvs, nbis = [], []
            for u in range(_U):
                v = row_v[pl.ds((i * _U + u) * _L, _L)]
                upd = v > bvs[u]
                nbvs.append(jnp.where(upd, v, bvs[u]))
                nbis.append(jnp.where(upd, jnp.full((_L,), i, jnp.int32), bis[u]))
            return tuple(nbvs), tuple(nbis)

        bvs, bis = lax.fori_loop(
            0, _NIT, step, ((neg,) * _U, (iz,) * _U)
        )

        bv = bvs[0]
        gi = (bis[0] * _U + 0) * _L + lane
        for u in range(1, _U):
            gu = (bis[u] * _U + u) * _L + lane
            upd = (bvs[u] > bv) | ((bvs[u] == bv) & (gu < gi))
            bv = jnp.where(upd, bvs[u], bv)
            gi = jnp.where(upd, gu, gi)

        for d in (1, 2, 4, 8):
            perm = lane ^ d
            ov = bv.at[perm].get(mode="promise_in_bounds")
            oi = gi.at[perm].get(mode="promise_in_bounds")
            upd = (ov > bv) | ((ov == bv) & (oi < gi))
            bv = jnp.where(upd, ov, bv)
            gi = jnp.where(upd, oi, gi)
        idx = lax.squeeze(lax.slice(gi, (0,), (1,)), dimensions=(0,))

        blk = idx // _L
        l = idx % _L
        oh_v[pl.ds(blk * _L, _L)] = jnp.where(l == lane, 1.0, 0.0).astype(
            jnp.float32
        )
        pltpu.sync_copy(oh_v, out_hbm.at[row])
        oh_v[pl.ds(blk * _L, _L)] = zeros16


@jax.jit
def kernel(coords):
    mesh = plsc.VectorSubcoreMesh(core_axis_name="c", subcore_axis_name="s")
    run = pl.kernel(
        _sc_body,
        mesh=mesh,
        out_type=jax.ShapeDtypeStruct((_B, _N), jnp.float32),
        scratch_types=[
            pltpu.VMEM((_N,), jnp.float32),
            pltpu.VMEM((_N,), jnp.float32),
        ],
    )
    return run(coords)


# speedup vs baseline: 1.2477x; 1.2477x over previous
import jax
import jax.numpy as jnp
from jax import lax
from jax.experimental import pallas as pl
from jax.experimental.pallas import tpu as pltpu
from jax.experimental.pallas import tpu_sc as plsc

_B = 128
_N = 8192
_L = 16
_NC = 2
_NS = 16
_NW = _NC * _NS
_RPW = _B // _NW
_U = 4
_NBLK = _N // _L
_NIT = _NBLK // _U


def _argmax_row(row_v, lane):
    neg = jnp.full((_L,), -jnp.inf, jnp.float32)
    iz = jnp.zeros((_L,), jnp.int32)

    def step(i, carry):
        bvs, bis = carry
        nbvs, nbis = [], []
        for u in range(_U):
            v = row_v[pl.ds((i * _U + u) * _L, _L)]
            upd = v > bvs[u]
            nbvs.append(jnp.where(upd, v, bvs[u]))
            nbis.append(jnp.where(upd, jnp.full((_L,), i, jnp.int32), bis[u]))
        return tuple(nbvs), tuple(nbis)

    bvs, bis = lax.fori_loop(
        0, _NIT, step, ((neg,) * _U, (iz,) * _U), unroll=4
    )

    bv = bvs[0]
    gi = bis[0] * (_U * _L) + lane
    for u in range(1, _U):
        gu = bis[u] * (_U * _L) + (u * _L) + lane
        upd = (bvs[u] > bv) | ((bvs[u] == bv) & (gu < gi))
        bv = jnp.where(upd, bvs[u], bv)
        gi = jnp.where(upd, gu, gi)

    for d in (1, 2, 4, 8):
        perm = lane ^ d
        ov = bv.at[perm].get(mode="promise_in_bounds")
        oi = gi.at[perm].get(mode="promise_in_bounds")
        upd = (ov > bv) | ((ov == bv) & (oi < gi))
        bv = jnp.where(upd, ov, bv)
        gi = jnp.where(upd, oi, gi)
    return lax.squeeze(lax.slice(gi, (0,), (1,)), dimensions=(0,))


def _sc_body(coords_hbm, out_hbm, r0, r1, r2, r3, oh_v, fix_v, in_sem, zo_sem):
    wid = lax.axis_index("s") * _NC + lax.axis_index("c")
    base = wid * _RPW
    lane = lax.broadcasted_iota(jnp.int32, (_L,), 0)
    rows_v = [r0, r1, r2, r3]

    in_copies = [
        pltpu.async_copy(coords_hbm.at[base + r], rows_v[r], in_sem)
        for r in range(_RPW)
    ]

    zeros16 = jnp.zeros((_L,), jnp.float32)

    def zinit(i, c):
        oh_v[pl.ds(i * _L, _L)] = zeros16
        return c

    lax.fori_loop(0, _NBLK, zinit, 0)

    zo_copies = [
        pltpu.async_copy(oh_v, out_hbm.at[base + r], zo_sem)
        for r in range(_RPW)
    ]

    for c in in_copies:
        c.wait()

    idxs = [_argmax_row(rows_v[r], lane) for r in range(_RPW)]

    for c in zo_copies:
        c.wait()

    for r in range(_RPW):
        idx = idxs[r]
        blk = idx // _L
        l = idx % _L
        fix_v[...] = jnp.where(l == lane, 1.0, 0.0).astype(jnp.float32)
        pltpu.sync_copy(fix_v, out_hbm.at[base + r, pl.ds(blk * _L, _L)])


@jax.jit
def kernel(coords):
    mesh = plsc.VectorSubcoreMesh(core_axis_name="c", subcore_axis_name="s")
    run = pl.kernel(
        _sc_body,
        mesh=mesh,
        out_type=jax.ShapeDtypeStruct((_B, _N), jnp.float32),
        scratch_types=[
            pltpu.VMEM((_N,), jnp.float32),
            pltpu.VMEM((_N,), jnp.float32),
            pltpu.VMEM((_N,), jnp.float32),
            pltpu.VMEM((_N,), jnp.float32),
            pltpu.VMEM((_N,), jnp.float32),
            pltpu.VMEM((_L,), jnp.float32),
            pltpu.SemaphoreType.DMA,
            pltpu.SemaphoreType.DMA,
        ],
    )
    return run(coords)


# speedup vs baseline: 6.8188x; 5.4652x over previous
import jax
import jax.numpy as jnp
from jax.experimental import pallas as pl

_B = 128
_N = 8192
_BR = 64


def _onehot_body(x_ref, o_ref):
    x = x_ref[...]
    idx = jnp.argmax(x, axis=1, keepdims=True)
    iota = jax.lax.broadcasted_iota(jnp.int32, x.shape, 1)
    o_ref[...] = (iota == idx).astype(jnp.float32)


def kernel(coords):
    return pl.pallas_call(
        _onehot_body,
        out_shape=jax.ShapeDtypeStruct((_B, _N), jnp.float32),
        grid=(_B // _BR,),
        in_specs=[pl.BlockSpec((_BR, _N), lambda i: (i, 0))],
        out_specs=pl.BlockSpec((_BR, _N), lambda i: (i, 0)),
    )(coords)
